# baseline (speedup 1.0000x reference)
import functools

import jax
import jax.numpy as jnp
from jax.experimental import pallas as pl
from jax.experimental.pallas import tpu as pltpu

_BF16 = jnp.bfloat16
_SQRT1_2 = 0.7071067811865476
_TB = (((1,), (1,)), ((), ()))


def _erf_approx(x):
    a1, a2, a3, a4, a5 = 0.254829592, -0.284496736, 1.421413741, -1.453152027, 1.061405429
    p = 0.3275911
    sgn = jnp.where(x >= 0, 1.0, -1.0).astype(x.dtype)
    ax = jnp.abs(x)
    t = 1.0 / (1.0 + p * ax)
    poly = ((((a5 * t + a4) * t + a3) * t + a2) * t + a1) * t
    return sgn * (1.0 - poly * jnp.exp(-ax * ax))


def _gelu(x):
    return 0.5 * x * (1.0 + _erf_approx(x * _SQRT1_2))


def _dot_t(a, bt):
    return jax.lax.dot_general(a, bt, _TB, preferred_element_type=jnp.float32)


def _reshape_rm(x, rows, cols):
    r0 = x.shape[0]
    flat = jnp.concatenate([x[i:i + 1, :] for i in range(r0)], axis=1)
    return jnp.concatenate([flat[:, i * cols:(i + 1) * cols] for i in range(rows)],
                           axis=0)


def _adj_compute(a, w11t, b11, w21t, b21, w12t, b12, w22t, b22, *, k):
    n_in = a.shape[0]
    h1 = w11t.shape[0]
    od = w22t.shape[0]

    h = _gelu(_dot_t(a, w11t) + b11)
    h = _reshape_rm(h, h1, n_in)
    h = _gelu(_dot_t(h, w12t) + b12)
    h = _dot_t(h, w22t) + b22
    h = _reshape_rm(h, od, h1)
    sc = _dot_t(h, w21t) + b21

    ex = jnp.exp(sc - jnp.max(sc, axis=0, keepdims=True))
    sm = ex / jnp.sum(ex, axis=0, keepdims=True)

    rid = jax.lax.broadcasted_iota(jnp.int32, sm.shape, 0)
    work = sm
    thr = jnp.full((1, sm.shape[1]), -jnp.inf, jnp.float32)
    for _ in range(k):
        thr = jnp.max(work, axis=0, keepdims=True)
        first = jnp.min(jnp.where(work == thr, rid, sm.shape[0]),
                        axis=0, keepdims=True)
        work = jnp.where(rid == first, -jnp.inf, work)
    return jnp.where(sm < thr, 0.0, sm)


def _fused_body(x_ref, w1t_ref, b1c_ref, w2t_ref, b2c_ref,
                wm1_ref, bm1_ref, wm2t_ref, bm2_ref,
                a_ref, w11t_ref, b11_ref, w21t_ref, b21_ref,
                w12t_ref, b12_ref, w22t_ref, b22_ref,
                z2_ref, adj_ref, acc_ref, *, k, om_dim, ot_dim, n_total):
    bb, T, n, s = x_ref.shape
    rows = bb * n

    hs = []
    for t in range(T):
        xt = x_ref[:, t, :, :].reshape(rows, s).astype(_BF16)
        hs.append(_dot_t(xt, w1t_ref[t].astype(_BF16)))
    h = jnp.concatenate(hs, axis=1) + b1c_ref[...]
    h = _gelu(h).astype(_BF16)
    e = _dot_t(h, w2t_ref[...].astype(_BF16)) + b2c_ref[...]
    h2 = _gelu(jnp.dot(e.astype(_BF16), wm1_ref[...].astype(_BF16),
                       preferred_element_type=jnp.float32) + bm1_ref[...])

    nc = om_dim * ot_dim
    jj = jax.lax.broadcasted_iota(jnp.int32, (nc, nc), 0)
    ii = jax.lax.broadcasted_iota(jnp.int32, (nc, nc), 1)
    ptf = ((jj % ot_dim) * om_dim + jj // ot_dim == ii).astype(jnp.float32)
    wm2p = jnp.dot(ptf.astype(_BF16), wm2t_ref[...].astype(_BF16),
                   preferred_element_type=jnp.float32).astype(_BF16)
    bm2p = _dot_t(bm2_ref[...], ptf)
    o = _dot_t(h2.astype(_BF16), wm2p) + bm2p

    i = pl.program_id(0)
    acc_ref[:, pl.ds(i * rows, rows)] = jnp.transpose(o, (1, 0))

    @pl.when(i == pl.num_programs(0) - 1)
    def _():
        for om in range(om_dim):
            a2 = jnp.transpose(acc_ref[om * ot_dim:(om + 1) * ot_dim, :],
                               (1, 0))
            a3 = a2.reshape(n_total // n, n, ot_dim)
            a4 = jnp.transpose(a3, (2, 0, 1))
            a5 = a4.reshape(ot_dim * (n_total // n), n)
            z2_ref[om * n:(om + 1) * n, :] = jnp.transpose(a5, (1, 0))
        adj_ref[...] = _adj_compute(
            a_ref[...], w11t_ref[...], b11_ref[...], w21t_ref[...],
            b21_ref[...], w12t_ref[...], b12_ref[...], w22t_ref[...],
            b22_ref[...], k=k)


def kernel(x, adj_matrix, type_w1, type_b1, type_w2, type_b2,
           adj_w1_d1, adj_b1_d1, adj_w2_d1, adj_b2_d1,
           adj_w1_d2, adj_b1_d2, adj_w2_d2, adj_b2_d2,
           main_w1, main_b1, main_w2, main_b2,
           w1_blk, b1_cat, w2_blk, b2_cat, wm1_x, bm1_x, wm2_x, bm2_x):
    B, T, N, S = x.shape
    Ot = type_w2.shape[2]
    Om = main_w2.shape[1]
    o_adj = adj_w2_d1.shape[1]

    w1t = jnp.transpose(type_w1, (0, 2, 1))
    w2t = w2_blk.T
    wm2t = wm2_x.T

    block_b = 16 if B % 16 == 0 else (2 if B % 2 == 0 else 1)
    grid = (B // block_b,)
    out_cols = wm2t.shape[0]

    in_specs = [pl.BlockSpec((block_b, T, N, S), lambda i: (i, 0, 0, 0))]
    weights = (w1t, b1_cat, w2t, b2_cat, wm1_x, bm1_x, wm2t, bm2_x,
               adj_matrix, adj_w1_d1.T, adj_b1_d1, adj_w2_d1.T, adj_b2_d1,
               adj_w1_d2.T, adj_b1_d2, adj_w2_d2.T, adj_b2_d2)
    for w in weights:
        nd = len(w.shape)
        in_specs.append(pl.BlockSpec(w.shape, lambda i, _nd=nd: (0,) * _nd))

    z2, adj = pl.pallas_call(
        functools.partial(_fused_body, k=4, om_dim=Om, ot_dim=Ot, n_total=B * N),
        out_shape=(jax.ShapeDtypeStruct((Om * N, Ot * B), jnp.float32),
                   jax.ShapeDtypeStruct((o_adj, o_adj), jnp.float32)),
        grid=grid,
        in_specs=in_specs,
        out_specs=(pl.BlockSpec((Om * N, Ot * B), lambda i: (0, 0)),
                   pl.BlockSpec((o_adj, o_adj), lambda i: (0, 0))),
        scratch_shapes=[pltpu.VMEM((out_cols, B * N), jnp.float32)],
        compiler_params=pltpu.CompilerParams(dimension_semantics=("arbitrary",)),
    )(x, *weights)

    y = z2.reshape(Om, N, Ot, B).transpose(3, 0, 1, 2)
    return y, adj


# speedup vs baseline: 1.0717x; 1.0717x over previous
import functools

import jax
import jax.numpy as jnp
from jax.experimental import pallas as pl
from jax.experimental.pallas import tpu as pltpu

_BF16 = jnp.bfloat16
_SQRT1_2 = 0.7071067811865476
_TB = (((1,), (1,)), ((), ()))


def _erf_approx(x):
    a1, a2, a3, a4, a5 = 0.254829592, -0.284496736, 1.421413741, -1.453152027, 1.061405429
    p = 0.3275911
    sgn = jnp.where(x >= 0, 1.0, -1.0).astype(x.dtype)
    ax = jnp.abs(x)
    t = 1.0 / (1.0 + p * ax)
    poly = ((((a5 * t + a4) * t + a3) * t + a2) * t + a1) * t
    return sgn * (1.0 - poly * jnp.exp(-ax * ax))


def _gelu(x):
    return 0.5 * x * (1.0 + _erf_approx(x * _SQRT1_2))


def _dot_t(a, bt):
    return jax.lax.dot_general(a, bt, _TB, preferred_element_type=jnp.float32)


def _reshape_rm(x, rows, cols):
    r0 = x.shape[0]
    flat = jnp.concatenate([x[i:i + 1, :] for i in range(r0)], axis=1)
    return jnp.concatenate([flat[:, i * cols:(i + 1) * cols] for i in range(rows)],
                           axis=0)


def _adj_compute(a, w11t, b11, w21t, b21, w12t, b12, w22t, b22, *, k):
    n_in = a.shape[0]
    h1 = w11t.shape[0]
    od = w22t.shape[0]

    h = _gelu(_dot_t(a, w11t) + b11)
    h = _reshape_rm(h, h1, n_in)
    h = _gelu(_dot_t(h, w12t) + b12)
    h = _dot_t(h, w22t) + b22
    h = _reshape_rm(h, od, h1)
    sc = _dot_t(h, w21t) + b21

    ex = jnp.exp(sc - jnp.max(sc, axis=0, keepdims=True))
    sm = ex / jnp.sum(ex, axis=0, keepdims=True)

    rid = jax.lax.broadcasted_iota(jnp.int32, sm.shape, 0)
    work = sm
    thr = jnp.full((1, sm.shape[1]), -jnp.inf, jnp.float32)
    for _ in range(k):
        thr = jnp.max(work, axis=0, keepdims=True)
        first = jnp.min(jnp.where(work == thr, rid, sm.shape[0]),
                        axis=0, keepdims=True)
        work = jnp.where(rid == first, -jnp.inf, work)
    return jnp.where(sm < thr, 0.0, sm)


def _fused_body(x_ref, w1t_ref, b1c_ref, w2t_ref, b2c_ref,
                wm1_ref, bm1_ref, wm2t_ref, bm2_ref,
                a_ref, w11t_ref, b11_ref, w21t_ref, b21_ref,
                w12t_ref, b12_ref, w22t_ref, b22_ref,
                o_ref, adj_ref, *, k):
    bb, T, n, s = x_ref.shape
    rows = bb * n

    hs = []
    for t in range(T):
        xt = x_ref[:, t, :, :].reshape(rows, s).astype(_BF16)
        hs.append(_dot_t(xt, w1t_ref[t].astype(_BF16)))
    h = jnp.concatenate(hs, axis=1) + b1c_ref[...]
    h = _gelu(h).astype(_BF16)
    e = _dot_t(h, w2t_ref[...].astype(_BF16)) + b2c_ref[...]
    h2 = _gelu(jnp.dot(e.astype(_BF16), wm1_ref[...].astype(_BF16),
                       preferred_element_type=jnp.float32) + bm1_ref[...])
    o = _dot_t(h2.astype(_BF16), wm2t_ref[...].astype(_BF16)) + bm2_ref[...]

    o_ref[...] = jnp.transpose(o, (1, 0))

    @pl.when(pl.program_id(0) == pl.num_programs(0) - 1)
    def _():
        adj_ref[...] = _adj_compute(
            a_ref[...], w11t_ref[...], b11_ref[...], w21t_ref[...],
            b21_ref[...], w12t_ref[...], b12_ref[...], w22t_ref[...],
            b22_ref[...], k=k)


def kernel(x, adj_matrix, type_w1, type_b1, type_w2, type_b2,
           adj_w1_d1, adj_b1_d1, adj_w2_d1, adj_b2_d1,
           adj_w1_d2, adj_b1_d2, adj_w2_d2, adj_b2_d2,
           main_w1, main_b1, main_w2, main_b2,
           w1_blk, b1_cat, w2_blk, b2_cat, wm1_x, bm1_x, wm2_x, bm2_x):
    B, T, N, S = x.shape
    Ot = type_w2.shape[2]
    Om = main_w2.shape[1]
    o_adj = adj_w2_d1.shape[1]

    w1t = jnp.transpose(type_w1, (0, 2, 1))
    w2t = w2_blk.T
    wm2t = wm2_x.T

    block_b = 16 if B % 16 == 0 else (2 if B % 2 == 0 else 1)
    grid = (B // block_b,)
    out_cols = wm2t.shape[0]

    in_specs = [pl.BlockSpec((block_b, T, N, S), lambda i: (i, 0, 0, 0))]
    weights = (w1t, b1_cat, w2t, b2_cat, wm1_x, bm1_x, wm2t, bm2_x,
               adj_matrix, adj_w1_d1.T, adj_b1_d1, adj_w2_d1.T, adj_b2_d1,
               adj_w1_d2.T, adj_b1_d2, adj_w2_d2.T, adj_b2_d2)
    for w in weights:
        nd = len(w.shape)
        in_specs.append(pl.BlockSpec(w.shape, lambda i, _nd=nd: (0,) * _nd))

    out_t, adj = pl.pallas_call(
        functools.partial(_fused_body, k=4),
        out_shape=(jax.ShapeDtypeStruct((out_cols, B * N), jnp.float32),
                   jax.ShapeDtypeStruct((o_adj, o_adj), jnp.float32)),
        grid=grid,
        in_specs=in_specs,
        out_specs=(pl.BlockSpec((out_cols, block_b * N), lambda i: (0, i)),
                   pl.BlockSpec((o_adj, o_adj), lambda i: (0, 0))),
        compiler_params=pltpu.CompilerParams(dimension_semantics=("arbitrary",)),
    )(x, *weights)

    y = out_t.reshape(Ot, Om, B, N).transpose(2, 1, 3, 0)
    return y, adj
